# Initial kernel scaffold; baseline (speedup 1.0000x reference)
import jax
import jax.numpy as jnp
from jax.experimental import pallas as pl


def kernel(x, Wr, Wsh, gsh, bsh, W1, g1, b1, W2):
    raise NotImplementedError("write your pallas kernel here")


# speedup vs baseline: 2.2805x; 2.2805x over previous
import jax
import jax.numpy as jnp
from jax.experimental import pallas as pl
from jax.experimental.pallas import tpu as pltpu

E = 8
TOPK = 2
EPS = 1e-5

_INTERPRET = False


def _route_body(x_ref, wr_ref, topi_ref, gates_ref):
    b = pl.program_id(0)
    xb = x_ref[0]
    pooled = jnp.mean(xb, axis=1, keepdims=True)
    logits = jax.lax.dot_general(
        pooled, wr_ref[...], (((0,), (1,)), ((), ())),
        preferred_element_type=jnp.float32)
    iota = jax.lax.broadcasted_iota(jnp.int32, (1, E), 1)
    m1 = jnp.max(logits)
    i1 = jnp.min(jnp.where(logits == m1, iota, E))
    masked = jnp.where(iota == i1, -jnp.inf, logits)
    m2 = jnp.max(masked)
    i2 = jnp.min(jnp.where(masked == m2, iota, E))
    g0 = jax.nn.sigmoid(m1 - m2)
    topi_ref[b, 0] = i1
    topi_ref[b, 1] = i2
    gates_ref[b, 0] = g0
    gates_ref[b, 1] = 1.0 - g0


def _moe_body(topi_s, gates_s, x_ref, wsh_ref, bsh_ref, w1_ref, b1_ref,
              w2_ref, out_ref):
    b = pl.program_id(0)
    k = pl.program_id(1)
    xb = x_ref[0]
    h = jnp.dot(w1_ref[0], xb, preferred_element_type=jnp.float32)
    h = h + b1_ref[0]
    h = h * jax.nn.sigmoid(h)
    o = jnp.dot(w2_ref[0], h, preferred_element_type=jnp.float32)
    contrib = gates_s[b, k] * o

    @pl.when(k == 0)
    def _():
        hs = jnp.dot(wsh_ref[...], xb, preferred_element_type=jnp.float32)
        hs = hs + bsh_ref[...]
        out_ref[0] = hs * jax.nn.sigmoid(hs) + contrib

    @pl.when(k != 0)
    def _():
        out_ref[0] += contrib


def kernel(x, Wr, Wsh, gsh, bsh, W1, g1, b1, W2):
    B, C, H, W = x.shape
    HW = H * W
    COUT = Wsh.shape[0]
    HID = W1.shape[1]
    x3 = x.reshape(B, C, HW)

    topi, gates = pl.pallas_call(
        _route_body,
        grid=(B,),
        in_specs=[
            pl.BlockSpec((1, C, HW), lambda b: (b, 0, 0)),
            pl.BlockSpec((E, C), lambda b: (0, 0)),
        ],
        out_specs=[
            pl.BlockSpec(memory_space=pltpu.SMEM),
            pl.BlockSpec(memory_space=pltpu.SMEM),
        ],
        out_shape=[
            jax.ShapeDtypeStruct((B, TOPK), jnp.int32),
            jax.ShapeDtypeStruct((B, TOPK), jnp.float32),
        ],
        interpret=_INTERPRET,
    )(x3, Wr)

    inv = 1.0 / jnp.sqrt(1.0 + EPS)
    Wshp = Wsh * (gsh * inv)[:, None]
    W1p = W1 * (g1 * inv)[:, :, None]
    bsh2 = bsh[:, None]
    b1r = b1[:, :, None]

    grid_spec = pltpu.PrefetchScalarGridSpec(
        num_scalar_prefetch=2,
        grid=(B, TOPK),
        in_specs=[
            pl.BlockSpec((1, C, HW), lambda b, k, ti, gs: (b, 0, 0)),
            pl.BlockSpec((COUT, C), lambda b, k, ti, gs: (0, 0)),
            pl.BlockSpec((COUT, 1), lambda b, k, ti, gs: (0, 0)),
            pl.BlockSpec((1, HID, C), lambda b, k, ti, gs: (ti[b, k], 0, 0)),
            pl.BlockSpec((1, HID, 1), lambda b, k, ti, gs: (ti[b, k], 0, 0)),
            pl.BlockSpec((1, COUT, HID), lambda b, k, ti, gs: (ti[b, k], 0, 0)),
        ],
        out_specs=pl.BlockSpec((1, COUT, HW), lambda b, k, ti, gs: (b, 0, 0)),
    )
    out = pl.pallas_call(
        _moe_body,
        grid_spec=grid_spec,
        out_shape=jax.ShapeDtypeStruct((B, COUT, HW), jnp.float32),
        interpret=_INTERPRET,
    )(topi, gates, x3, Wshp, bsh2, W1p, b1r, W2)

    return out.reshape(B, COUT, H, W)
